# baseline (speedup 1.0000x reference)
import jax
import jax.numpy as jnp
from jax import lax
from jax.experimental import pallas as pl
from jax.experimental.pallas import tpu as pltpu
from jax.experimental.pallas import tpu_sc as plsc

N = 16384
D = 128
L = 16
NS = 16
CHUNK = N // NS
NVEC = CHUNK // L
ROWS_PER_RND = 128
POS_PER_W = N // NS
NROUND = POS_PER_W // ROWS_PER_RND
MARGIN_THRESHOLD = 0.5


def _sc_body(proj_hbm, pair_hbm, out_hbm,
             H, BB, ST, PERM, PS,
             keys_v, bb_v, cstage_v, bb_local_v, totals_v, st_v, st2_v,
             ps2_v, pos2d_v, vals2d_v, eidx_v,
             rows0_v, rows1_v, rows2_v, rows3_v, row_v,
             sem0, sem1, sem2, sem3):
  c = lax.axis_index("c")
  s = lax.axis_index("s")
  iota = lax.iota(jnp.int32, L)
  lane15 = iota == (L - 1)

  pltpu.async_copy(pair_hbm.at[pl.ds(CHUNK * s, CHUNK)], keys_v, sem0)

  def vals_body(g, _):
    for m in range(8):
      vals2d_v[g, pl.ds(L * m, L)] = CHUNK * s + 128 * g + L * m + iota
    return 0
  lax.fori_loop(0, 8, vals_body, 0)

  def zero_body(i, _):
    for u in range(8):
      cstage_v[pl.ds(L * 8 * i + L * u, L)] = jnp.zeros((L,), jnp.int32)
    return 0
  lax.fori_loop(0, N // (L * 8), zero_body, 0)
  pltpu.make_async_copy(pair_hbm.at[pl.ds(CHUNK * s, CHUNK)], keys_v,
                        sem0).wait()

  def hist_body(j, _):
    k = keys_v[pl.ds(L * j, L)]
    cnt, last = plsc.scan_count(k)
    plsc.addupdate_scatter(cstage_v, [k], cnt, mask=last)
    return 0
  lax.fori_loop(0, NVEC, hist_body, 0)

  pltpu.sync_copy(cstage_v, H.at[s])
  plsc.subcore_barrier()

  for t in range(NS):
    pltpu.async_copy(H.at[t, pl.ds(CHUNK * s, CHUNK)],
                     cstage_v.at[pl.ds(CHUNK * t, CHUNK)], sem0)
  for t in range(NS):
    pltpu.make_async_copy(H.at[t, pl.ds(CHUNK * s, CHUNK)],
                          cstage_v.at[pl.ds(CHUNK * t, CHUNK)], sem0).wait()

  def colscan_body(j, _):
    acc = jnp.zeros((L,), jnp.int32)
    for t in range(NS):
      sl = pl.ds(CHUNK * t + L * j, L)
      tmp = cstage_v[sl]
      cstage_v[sl] = acc
      acc = acc + tmp
    cs = plsc.cumsum(acc)
    bb_local_v[pl.ds(L * j, L)] = cs - acc
    plsc.store_scatter(totals_v, [jnp.full((L,), j, jnp.int32)], cs,
                       mask=lane15)
    return 0
  lax.fori_loop(0, NVEC, colscan_body, 0)

  sum4 = (totals_v[pl.ds(0, L)] + totals_v[pl.ds(L, L)]
          + totals_v[pl.ds(2 * L, L)] + totals_v[pl.ds(3 * L, L)])
  st_v[0, :] = plsc.cumsum(sum4)
  pltpu.sync_copy(st_v.at[0], ST.at[s])
  for t in range(NS):
    pltpu.async_copy(cstage_v.at[pl.ds(CHUNK * t, CHUNK)],
                     H.at[t, pl.ds(CHUNK * s, CHUNK)], sem0)
  for t in range(NS):
    pltpu.make_async_copy(cstage_v.at[pl.ds(CHUNK * t, CHUNK)],
                          H.at[t, pl.ds(CHUNK * s, CHUNK)], sem0).wait()
  plsc.subcore_barrier()

  pltpu.sync_copy(ST, st2_v)
  stv = plsc.load_gather(st2_v, [iota, jnp.full((L,), L - 1, jnp.int32)])
  prior = jnp.sum(jnp.where(iota < s, stv, 0))

  carry = prior
  for q in range(NVEC // L):
    v = totals_v[pl.ds(L * q, L)]
    cs = plsc.cumsum(v)
    totals_v[pl.ds(L * q, L)] = cs - v + carry
    carry = carry + cs[L - 1]

  def binbase_body(j, _):
    sl = pl.ds(L * j, L)
    t0 = totals_v[pl.ds(j, L)][0]
    bb_local_v[sl] = bb_local_v[sl] + t0
    return 0
  lax.fori_loop(0, NVEC, binbase_body, 0)
  pltpu.sync_copy(bb_local_v, BB.at[pl.ds(CHUNK * s, CHUNK)])
  plsc.subcore_barrier()

  pltpu.async_copy(BB, bb_v, sem0)
  pltpu.async_copy(H.at[s], cstage_v, sem1)
  pltpu.make_async_copy(BB, bb_v, sem0).wait()
  pltpu.make_async_copy(H.at[s], cstage_v, sem1).wait()

  def rank_body(j, _):
    k = keys_v[pl.ds(L * j, L)]
    cnt, last = plsc.scan_count(k)
    base = plsc.load_gather(bb_v, [k]) + plsc.load_gather(cstage_v, [k])
    pos2d_v[j // 8, pl.ds(L * (j % 8), L)] = base + cnt - 1
    plsc.addupdate_scatter(cstage_v, [k], cnt, mask=last)
    return 0
  lax.fori_loop(0, NVEC, rank_body, 0)

  for g in range(8):
    pltpu.async_copy(vals2d_v.at[g], PERM.at[pos2d_v.at[g]], sem0)
  for g in range(8):
    pltpu.make_async_copy(vals2d_v.at[g], PERM.at[pos2d_v.at[g]],
                          sem0).wait()
  plsc.subcore_barrier()

  base_pos = POS_PER_W * s
  pltpu.sync_copy(PERM.at[pl.ds(base_pos, POS_PER_W)], eidx_v)
  bufs = (rows0_v, rows1_v, rows2_v, rows3_v)
  sems = (sem0, sem1, sem2, sem3)

  HALF = ROWS_PER_RND // 2

  def start(g, b):
    pltpu.async_copy(proj_hbm.at[eidx_v.at[pl.ds(ROWS_PER_RND * g, HALF)]],
                     bufs[b].at[pl.ds(0, HALF)], sems[b])
    pltpu.async_copy(
        proj_hbm.at[eidx_v.at[pl.ds(ROWS_PER_RND * g + HALF, HALF)]],
        bufs[b].at[pl.ds(HALF, HALF)], sems[b])

  def wait(g, b):
    pltpu.make_async_copy(
        proj_hbm.at[eidx_v.at[pl.ds(ROWS_PER_RND * g, HALF)]],
        bufs[b].at[pl.ds(0, HALF)], sems[b]).wait()
    pltpu.make_async_copy(
        proj_hbm.at[eidx_v.at[pl.ds(ROWS_PER_RND * g + HALF, HALF)]],
        bufs[b].at[pl.ds(HALF, HALF)], sems[b]).wait()

  def make_pair_body(buf):
    def pair_body(p, acc_vec):
      a = 2 * p
      accv = buf[a, pl.ds(0, L)] * buf[a + 1, pl.ds(0, L)]
      for m in range(1, D // L):
        accv = accv + buf[a, pl.ds(L * m, L)] * buf[a + 1, pl.ds(L * m, L)]
      cs = plsc.cumsum(accv)
      relu = jnp.maximum(cs - MARGIN_THRESHOLD, 0.0)
      return acc_vec + jnp.where(lane15, relu, 0.0)
    return pair_body

  start(0, 0)
  start(1, 1)
  start(2, 2)
  acc_vec = jnp.zeros((L,), jnp.float32)

  def round_body(i, acc_vec):
    for b in range(4):
      g = 4 * i + b
      wait(g, b)
      acc_vec = lax.fori_loop(0, ROWS_PER_RND // 2, make_pair_body(bufs[b]),
                              acc_vec)

      @pl.when(g + 3 < NROUND)
      def _():
        start(g + 3, (b + 3) % 4)
    return acc_vec
  acc_vec = lax.fori_loop(0, NROUND // 4, round_body, acc_vec)

  st_v[0, :] = plsc.bitcast(acc_vec, jnp.int32)
  pltpu.sync_copy(st_v.at[0], PS.at[s])
  plsc.subcore_barrier()

  @pl.when(s == 0)
  def _():
    pltpu.sync_copy(PS, ps2_v)
    pv = plsc.load_gather(ps2_v, [iota, jnp.full((L,), L - 1, jnp.int32)])
    total = jnp.sum(plsc.bitcast(pv, jnp.float32)) * (1.0 / (N // 2))
    row_v[...] = jnp.full((L,), total, jnp.float32)
    pltpu.sync_copy(row_v, out_hbm.at[0])


def _make_kernel():
  mesh = plsc.VectorSubcoreMesh(core_axis_name="c", subcore_axis_name="s",
                                num_cores=1)
  return pl.kernel(
      _sc_body,
      out_type=jax.ShapeDtypeStruct((1, L), jnp.float32),
      mesh=mesh,
      compiler_params=pltpu.CompilerParams(needs_layout_passes=False),
      scratch_types=[
          pltpu.VMEM_SHARED((NS, N), jnp.int32),
          pltpu.VMEM_SHARED((N,), jnp.int32),
          pltpu.VMEM_SHARED((NS, L), jnp.int32),
          pltpu.VMEM_SHARED((N,), jnp.int32),
          pltpu.VMEM_SHARED((NS, L), jnp.int32),
          pltpu.VMEM((CHUNK,), jnp.int32),
          pltpu.VMEM((N,), jnp.int32),
          pltpu.VMEM((N,), jnp.int32),
          pltpu.VMEM((CHUNK,), jnp.int32),
          pltpu.VMEM((NVEC + L,), jnp.int32),
          pltpu.VMEM((1, L), jnp.int32),
          pltpu.VMEM((NS, L), jnp.int32),
          pltpu.VMEM((NS, L), jnp.int32),
          pltpu.VMEM((8, 128), jnp.int32),
          pltpu.VMEM((8, 128), jnp.int32),
          pltpu.VMEM((POS_PER_W,), jnp.int32),
          pltpu.VMEM((ROWS_PER_RND, D), jnp.float32),
          pltpu.VMEM((ROWS_PER_RND, D), jnp.float32),
          pltpu.VMEM((ROWS_PER_RND, D), jnp.float32),
          pltpu.VMEM((ROWS_PER_RND, D), jnp.float32),
          pltpu.VMEM((L,), jnp.float32),
          pltpu.SemaphoreType.DMA,
          pltpu.SemaphoreType.DMA,
          pltpu.SemaphoreType.DMA,
          pltpu.SemaphoreType.DMA,
      ],
  )


def kernel(projections, labels, pair_indices):
  del labels
  out = _make_kernel()(projections, pair_indices.astype(jnp.int32))
  return out[0, 0]


# speedup vs baseline: 1.0046x; 1.0046x over previous
import jax
import jax.numpy as jnp
from jax import lax
from jax.experimental import pallas as pl
from jax.experimental.pallas import tpu as pltpu
from jax.experimental.pallas import tpu_sc as plsc

N = 16384
D = 128
L = 16
NS = 16
CHUNK = N // NS
NVEC = CHUNK // L
ROWS_PER_RND = 128
POS_PER_W = N // NS
NROUND = POS_PER_W // ROWS_PER_RND
MARGIN_THRESHOLD = 0.5


def _sc_body(proj_hbm, pair_hbm, out_hbm,
             H, BB, ST, PERM, PS,
             keys_v, bb_v, cstage_v, bb_local_v, totals_v, st_v, st2_v,
             ps2_v, pos2d_v, vals2d_v, eidx_v,
             rows0_v, rows1_v, rows2_v, rows3_v, row_v,
             sem0, sem1, sem2, sem3):
  c = lax.axis_index("c")
  s = lax.axis_index("s")
  iota = lax.iota(jnp.int32, L)
  lane15 = iota == (L - 1)

  pltpu.async_copy(pair_hbm.at[pl.ds(CHUNK * s, CHUNK)], keys_v, sem0)

  def vals_body(g, _):
    for m in range(8):
      vals2d_v[g, pl.ds(L * m, L)] = CHUNK * s + 128 * g + L * m + iota
    return 0
  lax.fori_loop(0, 8, vals_body, 0)

  def zero_body(i, _):
    for u in range(8):
      cstage_v[pl.ds(L * 8 * i + L * u, L)] = jnp.zeros((L,), jnp.int32)
    return 0
  lax.fori_loop(0, N // (L * 8), zero_body, 0)
  pltpu.make_async_copy(pair_hbm.at[pl.ds(CHUNK * s, CHUNK)], keys_v,
                        sem0).wait()

  def hist_body(j, _):
    k = keys_v[pl.ds(L * j, L)]
    cnt, last = plsc.scan_count(k)
    plsc.addupdate_scatter(cstage_v, [k], cnt, mask=last)
    return 0
  lax.fori_loop(0, NVEC, hist_body, 0)

  pltpu.sync_copy(cstage_v, H.at[s])
  plsc.subcore_barrier()

  for t in range(NS):
    pltpu.async_copy(H.at[t, pl.ds(CHUNK * s, CHUNK)],
                     cstage_v.at[pl.ds(CHUNK * t, CHUNK)], sem0)
  for t in range(NS):
    pltpu.make_async_copy(H.at[t, pl.ds(CHUNK * s, CHUNK)],
                          cstage_v.at[pl.ds(CHUNK * t, CHUNK)], sem0).wait()

  def colscan_body(j, _):
    acc = jnp.zeros((L,), jnp.int32)
    for t in range(NS):
      sl = pl.ds(CHUNK * t + L * j, L)
      tmp = cstage_v[sl]
      cstage_v[sl] = acc
      acc = acc + tmp
    cs = plsc.cumsum(acc)
    bb_local_v[pl.ds(L * j, L)] = cs - acc
    plsc.store_scatter(totals_v, [jnp.full((L,), j, jnp.int32)], cs,
                       mask=lane15)
    return 0
  lax.fori_loop(0, NVEC, colscan_body, 0)

  sum4 = (totals_v[pl.ds(0, L)] + totals_v[pl.ds(L, L)]
          + totals_v[pl.ds(2 * L, L)] + totals_v[pl.ds(3 * L, L)])
  st_v[0, :] = plsc.cumsum(sum4)
  pltpu.sync_copy(st_v.at[0], ST.at[s])
  for t in range(NS):
    pltpu.async_copy(cstage_v.at[pl.ds(CHUNK * t, CHUNK)],
                     H.at[t, pl.ds(CHUNK * s, CHUNK)], sem0)
  for t in range(NS):
    pltpu.make_async_copy(cstage_v.at[pl.ds(CHUNK * t, CHUNK)],
                          H.at[t, pl.ds(CHUNK * s, CHUNK)], sem0).wait()
  plsc.subcore_barrier()

  pltpu.sync_copy(ST, st2_v)
  stv = plsc.load_gather(st2_v, [iota, jnp.full((L,), L - 1, jnp.int32)])
  prior = jnp.sum(jnp.where(iota < s, stv, 0))

  carry = prior
  for q in range(NVEC // L):
    v = totals_v[pl.ds(L * q, L)]
    cs = plsc.cumsum(v)
    totals_v[pl.ds(L * q, L)] = cs - v + carry
    carry = carry + cs[L - 1]

  def binbase_body(j, _):
    sl = pl.ds(L * j, L)
    t0 = totals_v[pl.ds(j, L)][0]
    bb_local_v[sl] = bb_local_v[sl] + t0
    return 0
  lax.fori_loop(0, NVEC, binbase_body, 0)
  pltpu.sync_copy(bb_local_v, BB.at[pl.ds(CHUNK * s, CHUNK)])
  plsc.subcore_barrier()

  pltpu.async_copy(BB, bb_v, sem0)
  pltpu.async_copy(H.at[s], cstage_v, sem1)
  pltpu.make_async_copy(BB, bb_v, sem0).wait()
  pltpu.make_async_copy(H.at[s], cstage_v, sem1).wait()

  def rank_body(j, _):
    k = keys_v[pl.ds(L * j, L)]
    cnt, last = plsc.scan_count(k)
    base = plsc.load_gather(bb_v, [k]) + plsc.load_gather(cstage_v, [k])
    pos2d_v[j // 8, pl.ds(L * (j % 8), L)] = base + cnt - 1
    plsc.addupdate_scatter(cstage_v, [k], cnt, mask=last)
    return 0
  lax.fori_loop(0, NVEC, rank_body, 0)

  for g in range(8):
    pltpu.async_copy(vals2d_v.at[g], PERM.at[pos2d_v.at[g]], sem0)
  for g in range(8):
    pltpu.make_async_copy(vals2d_v.at[g], PERM.at[pos2d_v.at[g]],
                          sem0).wait()
  plsc.subcore_barrier()

  base_pos = POS_PER_W * s
  pltpu.sync_copy(PERM.at[pl.ds(base_pos, POS_PER_W)], eidx_v)
  bufs = (rows0_v, rows1_v, rows2_v, rows3_v)
  sems = (sem0, sem1, sem2, sem3)

  def start(g, b):
    pltpu.async_copy(proj_hbm.at[eidx_v.at[pl.ds(ROWS_PER_RND * g,
                                                 ROWS_PER_RND)]],
                     bufs[b], sems[b])

  def wait(g, b):
    pltpu.make_async_copy(proj_hbm.at[eidx_v.at[pl.ds(ROWS_PER_RND * g,
                                                      ROWS_PER_RND)]],
                          bufs[b], sems[b]).wait()

  def make_pair_body(buf):
    def pair_body(p, acc_vec):
      a = 2 * p
      accv = buf[a, pl.ds(0, L)] * buf[a + 1, pl.ds(0, L)]
      for m in range(1, D // L):
        accv = accv + buf[a, pl.ds(L * m, L)] * buf[a + 1, pl.ds(L * m, L)]
      cs = plsc.cumsum(accv)
      relu = jnp.maximum(cs - MARGIN_THRESHOLD, 0.0)
      return acc_vec + jnp.where(lane15, relu, 0.0)
    return pair_body

  start(0, 0)
  start(1, 1)
  start(2, 2)
  acc_vec = jnp.zeros((L,), jnp.float32)

  def round_body(i, acc_vec):
    for b in range(4):
      g = 4 * i + b
      wait(g, b)
      acc_vec = lax.fori_loop(0, ROWS_PER_RND // 2, make_pair_body(bufs[b]),
                              acc_vec)

      @pl.when(g + 3 < NROUND)
      def _():
        start(g + 3, (b + 3) % 4)
    return acc_vec
  acc_vec = lax.fori_loop(0, NROUND // 4, round_body, acc_vec)

  st_v[0, :] = plsc.bitcast(acc_vec, jnp.int32)
  pltpu.sync_copy(st_v.at[0], PS.at[s])
  plsc.subcore_barrier()

  @pl.when(s == 0)
  def _():
    pltpu.sync_copy(PS, ps2_v)
    pv = plsc.load_gather(ps2_v, [iota, jnp.full((L,), L - 1, jnp.int32)])
    total = jnp.sum(plsc.bitcast(pv, jnp.float32)) * (1.0 / (N // 2))
    row_v[...] = jnp.full((L,), total, jnp.float32)
    pltpu.sync_copy(row_v, out_hbm.at[0])


def _make_kernel():
  mesh = plsc.VectorSubcoreMesh(core_axis_name="c", subcore_axis_name="s",
                                num_cores=1)
  return pl.kernel(
      _sc_body,
      out_type=jax.ShapeDtypeStruct((1, L), jnp.float32),
      mesh=mesh,
      compiler_params=pltpu.CompilerParams(needs_layout_passes=False),
      scratch_types=[
          pltpu.VMEM_SHARED((NS, N), jnp.int32),
          pltpu.VMEM_SHARED((N,), jnp.int32),
          pltpu.VMEM_SHARED((NS, L), jnp.int32),
          pltpu.VMEM_SHARED((N,), jnp.int32),
          pltpu.VMEM_SHARED((NS, L), jnp.int32),
          pltpu.VMEM((CHUNK,), jnp.int32),
          pltpu.VMEM((N,), jnp.int32),
          pltpu.VMEM((N,), jnp.int32),
          pltpu.VMEM((CHUNK,), jnp.int32),
          pltpu.VMEM((NVEC + L,), jnp.int32),
          pltpu.VMEM((1, L), jnp.int32),
          pltpu.VMEM((NS, L), jnp.int32),
          pltpu.VMEM((NS, L), jnp.int32),
          pltpu.VMEM((8, 128), jnp.int32),
          pltpu.VMEM((8, 128), jnp.int32),
          pltpu.VMEM((POS_PER_W,), jnp.int32),
          pltpu.VMEM((ROWS_PER_RND, D), jnp.float32),
          pltpu.VMEM((ROWS_PER_RND, D), jnp.float32),
          pltpu.VMEM((ROWS_PER_RND, D), jnp.float32),
          pltpu.VMEM((ROWS_PER_RND, D), jnp.float32),
          pltpu.VMEM((L,), jnp.float32),
          pltpu.SemaphoreType.DMA,
          pltpu.SemaphoreType.DMA,
          pltpu.SemaphoreType.DMA,
          pltpu.SemaphoreType.DMA,
      ],
  )


def kernel(projections, labels, pair_indices):
  del labels
  out = _make_kernel()(projections, pair_indices.astype(jnp.int32))
  return out[0, 0]
